# baseline (speedup 1.0000x reference)
import functools

import jax
import jax.numpy as jnp
from jax import lax
from jax.experimental import pallas as pl
from jax.experimental.pallas import tpu as pltpu
from jax.experimental.pallas import tpu_sc as plsc

from math import gcd as _gcd

_CHUNK = 80
_NTILES = 16
_NSC = 2
_NBUF = 3
_NIDX = 6


def _ranged_copy(src_ref, dst_ref, r0, rpt, tail0, rem, is_last):
    pltpu.sync_copy(src_ref.at[pl.ds(r0, rpt)], dst_ref.at[pl.ds(r0, rpt)])
    if rem:
        @pl.when(is_last)
        def _():
            pltpu.sync_copy(src_ref.at[pl.ds(tail0, rem)],
                            dst_ref.at[pl.ds(tail0, rem)])


def _agg_body(n_nodes, n_chunks,
              t, src, dst, zacc,
              pL, pR, sslot, dslot, rows_v, acc_sh, gsem, isem, ssem):
    cid = lax.axis_index("c")
    sid = lax.axis_index("s")
    wid = cid * _NTILES + sid
    rpt = (n_nodes // _NTILES) // 8 * 8
    tail0 = _NTILES * rpt
    rem = n_nodes - tail0
    r0 = sid * rpt
    is_last = sid == _NTILES - 1

    _ranged_copy(zacc, acc_sh, r0, rpt, tail0, rem, is_last)
    plsc.subcore_barrier()

    def load_idx(cc, j):
        pltpu.async_copy(src.at[wid, cc], sslot.at[j], isem.at[j])
        pltpu.async_copy(dst.at[wid, cc], dslot.at[j], isem.at[j])

    def wait_idx(cc, j):
        pltpu.make_async_copy(src.at[wid, cc], sslot.at[j], isem.at[j]).wait()
        pltpu.make_async_copy(dst.at[wid, cc], dslot.at[j], isem.at[j]).wait()

    for j in range(_NIDX):
        load_idx(j, j)
    for b in range(_NBUF):
        wait_idx(b, b)
        pltpu.async_copy(t.at[sslot.at[b, 0]], rows_v.at[b], gsem.at[b])

    steps_per_group = _NIDX * _NBUF // _gcd(_NIDX, _NBUF)
    n_groups = -(-n_chunks // steps_per_group)

    def ring(g, carry):
        for k in range(steps_per_group):
            c = g * steps_per_group + k
            b = k % _NBUF
            j = k % _NIDX
            b1 = (k - 1) % _NBUF
            j1 = (k - 1) % _NIDX
            jn1 = (k - 1 + _NBUF) % _NIDX

            @pl.when(c < n_chunks)
            def _():
                pltpu.make_async_copy(t.at[sslot.at[j, 0]], rows_v.at[b],
                                      gsem.at[b]).wait()
                pltpu.async_copy(rows_v.at[b], acc_sh.at[dslot.at[j, 0]],
                                 ssem.at[b], add=True)

            @pl.when(jnp.logical_and(c >= 1, c - 1 + _NBUF < n_chunks))
            def _():
                pltpu.make_async_copy(rows_v.at[b1],
                                      acc_sh.at[dslot.at[j1, 0]],
                                      ssem.at[b1]).wait()
                wait_idx(c - 1 + _NBUF, jn1)
                pltpu.async_copy(t.at[sslot.at[jn1, 0]], rows_v.at[b1],
                                 gsem.at[b1])

            @pl.when(jnp.logical_and(c >= 1, c - 1 + _NIDX < n_chunks))
            def _():
                load_idx(c - 1 + _NIDX, j1)
        return carry

    lax.fori_loop(0, n_groups, ring, 0)

    for b in range(_NBUF):
        pltpu.make_async_copy(rows_v.at[b], acc_sh.at[dslot.at[0, 0]],
                              ssem.at[b]).wait()
    plsc.subcore_barrier()

    @pl.when(cid == 0)
    def _():
        _ranged_copy(acc_sh, pL, r0, rpt, tail0, rem, is_last)

    @pl.when(cid == 1)
    def _():
        _ranged_copy(acc_sh, pR, r0, rpt, tail0, rem, is_last)


def _deg_body(n_pad, n_chunks, h,
              dst, dgA, dgB,
              dst_v, hist_v, sbuf, outbuf, bbuf, deg_sh, sem):
    cid = lax.axis_index("c")
    sid = lax.axis_index("s")
    wid = cid * _NTILES + sid
    seg = n_pad // _NTILES
    f32 = jnp.float32
    zero16 = jnp.zeros((16,), f32)
    izero16 = jnp.zeros((16,), jnp.int32)
    ones16 = jnp.ones((16,), f32)

    pltpu.sync_copy(dst.at[wid], dst_v)

    def zhist(i, carry):
        hist_v[0, pl.ds(i * 16, 16)] = zero16
        return carry

    lax.fori_loop(0, n_pad // 16, zhist, 0)

    def count(c, carry):
        for k in range(_CHUNK // 16):
            iv = dst_v[c, pl.ds(k * 16, 16)]
            plsc.addupdate_scatter(hist_v, [izero16, iv], ones16)
        return carry

    lax.fori_loop(0, n_chunks, count, 0)

    pltpu.sync_copy(hist_v, deg_sh.at[sid])
    plsc.subcore_barrier()

    for r in range(_NTILES):
        pltpu.sync_copy(deg_sh.at[r, 0, pl.ds(sid * seg, seg)], sbuf.at[r])

    def comb(i, carry):
        s = sbuf[0, pl.ds(i * 16, 16)]
        for r in range(1, _NTILES):
            s = s + sbuf[r, pl.ds(i * 16, 16)]
        outbuf[pl.ds(i * 16, 16)] = s
        return carry

    lax.fori_loop(0, seg // 16, comb, 0)

    def slab(s_, carry):
        def row(r, carry2):
            ridx = jnp.full((16,), s_ * 80 + r, jnp.int32)
            v = plsc.load_gather(outbuf, [ridx])
            for j in range(h // 16):
                bbuf[r, pl.ds(j * 16, 16)] = v
            return carry2

        lax.fori_loop(0, 80, row, 0)
        base = sid * seg + s_ * 80

        @pl.when(cid == 0)
        def _():
            pltpu.sync_copy(bbuf, dgA.at[pl.ds(base, 80)])

        @pl.when(cid == 1)
        def _():
            pltpu.sync_copy(bbuf, dgB.at[pl.ds(base, 80)])

        return carry

    lax.fori_loop(0, seg // 80, slab, 0)


def _make_agg(n_nodes, h, n_chunks):
    f32 = jnp.float32
    outs = [jax.ShapeDtypeStruct((n_nodes, h), f32),
            jax.ShapeDtypeStruct((n_nodes, h), f32)]
    scratch = [
        pltpu.VMEM((_NIDX, 1, _CHUNK), jnp.int32),
        pltpu.VMEM((_NIDX, 1, _CHUNK), jnp.int32),
        pltpu.VMEM((_NBUF, _CHUNK, h), f32),
        pltpu.VMEM_SHARED((n_nodes, h), f32),
        pltpu.SemaphoreType.DMA((_NBUF,)),
        pltpu.SemaphoreType.DMA((_NIDX,)),
        pltpu.SemaphoreType.DMA((_NBUF,)),
    ]
    mesh = plsc.VectorSubcoreMesh(core_axis_name="c", subcore_axis_name="s")
    return pl.kernel(
        functools.partial(_agg_body, n_nodes, n_chunks),
        out_type=tuple(outs), mesh=mesh, scratch_types=scratch,
    )


def _make_deg(n_pad, h, n_chunks):
    f32 = jnp.float32
    seg = n_pad // _NTILES
    outs = [jax.ShapeDtypeStruct((n_pad, h), f32),
            jax.ShapeDtypeStruct((n_pad, h), f32)]
    scratch = [
        pltpu.VMEM((n_chunks, _CHUNK), jnp.int32),
        pltpu.VMEM((1, n_pad), f32),
        pltpu.VMEM((_NTILES, seg), f32),
        pltpu.VMEM((seg,), f32),
        pltpu.VMEM((80, h), f32),
        pltpu.VMEM_SHARED((_NTILES, 1, n_pad), f32),
        pltpu.SemaphoreType.DMA,
    ]
    mesh = plsc.VectorSubcoreMesh(core_axis_name="c", subcore_axis_name="s")
    return pl.kernel(
        functools.partial(_deg_body, n_pad, n_chunks, h),
        out_type=tuple(outs), mesh=mesh, scratch_types=scratch,
        compiler_params=pltpu.CompilerParams(needs_layout_passes=False),
    )


def _l0_body(x_ref, ws_ref, wn_ref, s0_ref, t0_ref):
    xb = x_ref[...]
    s0_ref[...] = jnp.dot(xb, ws_ref[...], preferred_element_type=jnp.float32)
    t0_ref[...] = jnp.dot(xb, wn_ref[...], preferred_element_type=jnp.float32)


def _mid_body(s0_ref, pL_ref, pR_ref, dA_ref, dB_ref,
              wst_ref, wsb_ref, wnt_ref, wnb_ref,
              s1_ref, t1_ref):
    rs = jnp.maximum(s0_ref[...], 0.0)
    deg = dA_ref[:, 0:1] + dB_ref[:, 0:1]
    inv = 1.0 / jnp.maximum(deg, 1.0)
    a = pL_ref[...] + pR_ref[...]
    ra = jnp.maximum(a, 0.0) * inv
    s1_ref[...] = (jnp.dot(rs, wst_ref[...], preferred_element_type=jnp.float32)
                   + jnp.dot(ra, wsb_ref[...], preferred_element_type=jnp.float32))
    t1_ref[...] = (jnp.dot(rs, wnt_ref[...], preferred_element_type=jnp.float32)
                   + jnp.dot(ra, wnb_ref[...], preferred_element_type=jnp.float32))


def _out_body(s1_ref, pL_ref, pR_ref, dA_ref, dB_ref,
              wot_ref, wob_ref, o_ref):
    rs = jnp.maximum(s1_ref[...], 0.0)
    deg = dA_ref[:, 0:1] + dB_ref[:, 0:1]
    inv = 1.0 / jnp.maximum(deg, 1.0)
    a = pL_ref[...] + pR_ref[...]
    ra = jnp.maximum(a, 0.0) * inv
    o_ref[...] = (jnp.dot(rs, wot_ref[...], preferred_element_type=jnp.float32)
                  + jnp.dot(ra, wob_ref[...], preferred_element_type=jnp.float32))


def _row_spec(rb, w):
    return pl.BlockSpec((rb, w), lambda i: (i, 0))


def _full_spec(hh, w):
    return pl.BlockSpec((hh, w), lambda i: (0, 0))


def kernel(x, edge_index, W0_self, W0_neigh, W1_self, W1_neigh, W_out):
    f32 = jnp.float32
    n, d = x.shape
    e = edge_index.shape[1]
    h = W0_self.shape[1]
    c = W_out.shape[1]
    nw = _NSC * _NTILES
    assert e % (nw * _CHUNK) == 0 and n % _NTILES == 0
    n_chunks = e // (nw * _CHUNK)

    src4 = edge_index[0].reshape(nw, n_chunks, 1, _CHUNK)
    dst4 = edge_index[1].reshape(nw, n_chunks, 1, _CHUNK)
    dst3 = edge_index[1].reshape(nw, n_chunks, _CHUNK)
    zacc = jnp.zeros((n, h), f32)

    rb = 1000 if n % 1000 == 0 else n // 8
    grid = n // rb

    s0, t0 = pl.pallas_call(
        _l0_body,
        grid=(grid,),
        in_specs=[_row_spec(rb, d), _full_spec(d, h), _full_spec(d, h)],
        out_specs=[_row_spec(rb, h), _row_spec(rb, h)],
        out_shape=[jax.ShapeDtypeStruct((n, h), f32),
                   jax.ShapeDtypeStruct((n, h), f32)],
    )(x, W0_self, W0_neigh)

    n_pad = -(-n // (_NTILES * 80)) * (_NTILES * 80)
    dgA, dgB = _make_deg(n_pad, h, n_chunks)(dst3)

    p0L, p0R = _make_agg(n, h, n_chunks)(t0, src4, dst4, zacc)

    s1, t1 = pl.pallas_call(
        _mid_body,
        grid=(grid,),
        in_specs=[_row_spec(rb, h), _row_spec(rb, h), _row_spec(rb, h),
                  _row_spec(rb, h), _row_spec(rb, h),
                  _full_spec(h, h), _full_spec(h, h),
                  _full_spec(h, h), _full_spec(h, h)],
        out_specs=[_row_spec(rb, h), _row_spec(rb, h)],
        out_shape=[jax.ShapeDtypeStruct((n, h), f32),
                   jax.ShapeDtypeStruct((n, h), f32)],
    )(s0, p0L, p0R, dgA, dgB,
      W1_self[:h], W1_self[h:], W1_neigh[:h], W1_neigh[h:])

    p1L, p1R = _make_agg(n, h, n_chunks)(t1, src4, dst4, zacc)

    out = pl.pallas_call(
        _out_body,
        grid=(grid,),
        in_specs=[_row_spec(rb, h), _row_spec(rb, h), _row_spec(rb, h),
                  _row_spec(rb, h), _row_spec(rb, h),
                  _full_spec(h, c), _full_spec(h, c)],
        out_specs=_row_spec(rb, c),
        out_shape=jax.ShapeDtypeStruct((n, c), f32),
    )(s1, p1L, p1R, dgA, dgB, W_out[:h], W_out[h:])

    return out


# speedup vs baseline: 1.0217x; 1.0217x over previous
import functools

import jax
import jax.numpy as jnp
from jax import lax
from jax.experimental import pallas as pl
from jax.experimental.pallas import tpu as pltpu
from jax.experimental.pallas import tpu_sc as plsc

from math import gcd as _gcd

_CHUNK = 80
_NTILES = 16
_NSC = 2
_NBUF = 3
_NIDX = 6


def _ranged_copy(src_ref, dst_ref, r0, rpt, tail0, rem, is_last):
    pltpu.sync_copy(src_ref.at[pl.ds(r0, rpt)], dst_ref.at[pl.ds(r0, rpt)])
    if rem:
        @pl.when(is_last)
        def _():
            pltpu.sync_copy(src_ref.at[pl.ds(tail0, rem)],
                            dst_ref.at[pl.ds(tail0, rem)])


def _agg_body(n_nodes, n_chunks,
              t, src, dst, zacc,
              pL, pR, sslot, dslot, rows_v, acc_sh, gsem, isem):
    cid = lax.axis_index("c")
    sid = lax.axis_index("s")
    wid = cid * _NTILES + sid
    rpt = (n_nodes // _NTILES) // 8 * 8
    tail0 = _NTILES * rpt
    rem = n_nodes - tail0
    r0 = sid * rpt
    is_last = sid == _NTILES - 1

    _ranged_copy(zacc, acc_sh, r0, rpt, tail0, rem, is_last)
    plsc.subcore_barrier()

    def load_idx(cc, j):
        pltpu.async_copy(src.at[wid, cc], sslot.at[j], isem.at[j])
        pltpu.async_copy(dst.at[wid, cc], dslot.at[j], isem.at[j])

    def wait_idx(cc, j):
        pltpu.make_async_copy(src.at[wid, cc], sslot.at[j], isem.at[j]).wait()
        pltpu.make_async_copy(dst.at[wid, cc], dslot.at[j], isem.at[j]).wait()

    for j in range(_NIDX):
        load_idx(j, j)
    for b in range(_NBUF):
        wait_idx(b, b)
        pltpu.async_copy(t.at[sslot.at[b, 0]], rows_v.at[b], gsem.at[b])

    steps_per_group = _NIDX * _NBUF // _gcd(_NIDX, _NBUF)
    n_groups = -(-n_chunks // steps_per_group)

    def ring(g, carry):
        for k in range(steps_per_group):
            c = g * steps_per_group + k
            b = k % _NBUF
            j = k % _NIDX

            @pl.when(c < n_chunks)
            def _():
                pltpu.make_async_copy(t.at[sslot.at[j, 0]], rows_v.at[b],
                                      gsem.at[b]).wait()
                pltpu.sync_copy(rows_v.at[b], acc_sh.at[dslot.at[j, 0]],
                                add=True)

            @pl.when(c + _NBUF < n_chunks)
            def _():
                jn = (k + _NBUF) % _NIDX
                wait_idx(c + _NBUF, jn)
                pltpu.async_copy(t.at[sslot.at[jn, 0]], rows_v.at[b],
                                 gsem.at[b])

            @pl.when(c + _NIDX < n_chunks)
            def _():
                load_idx(c + _NIDX, j)
        return carry

    lax.fori_loop(0, n_groups, ring, 0)
    plsc.subcore_barrier()

    @pl.when(cid == 0)
    def _():
        _ranged_copy(acc_sh, pL, r0, rpt, tail0, rem, is_last)

    @pl.when(cid == 1)
    def _():
        _ranged_copy(acc_sh, pR, r0, rpt, tail0, rem, is_last)


def _deg_body(n_pad, n_chunks, h,
              dst, dgA, dgB,
              dst_v, hist_v, sbuf, outbuf, bbuf, deg_sh, sem):
    cid = lax.axis_index("c")
    sid = lax.axis_index("s")
    wid = cid * _NTILES + sid
    seg = n_pad // _NTILES
    f32 = jnp.float32
    zero16 = jnp.zeros((16,), f32)
    izero16 = jnp.zeros((16,), jnp.int32)
    ones16 = jnp.ones((16,), f32)

    pltpu.sync_copy(dst.at[wid], dst_v)

    def zhist(i, carry):
        hist_v[0, pl.ds(i * 16, 16)] = zero16
        return carry

    lax.fori_loop(0, n_pad // 16, zhist, 0)

    def count(c, carry):
        for k in range(_CHUNK // 16):
            iv = dst_v[c, pl.ds(k * 16, 16)]
            plsc.addupdate_scatter(hist_v, [izero16, iv], ones16)
        return carry

    lax.fori_loop(0, n_chunks, count, 0)

    pltpu.sync_copy(hist_v, deg_sh.at[sid])
    plsc.subcore_barrier()

    for r in range(_NTILES):
        pltpu.sync_copy(deg_sh.at[r, 0, pl.ds(sid * seg, seg)], sbuf.at[r])

    def comb(i, carry):
        s = sbuf[0, pl.ds(i * 16, 16)]
        for r in range(1, _NTILES):
            s = s + sbuf[r, pl.ds(i * 16, 16)]
        outbuf[pl.ds(i * 16, 16)] = s
        return carry

    lax.fori_loop(0, seg // 16, comb, 0)

    def slab(s_, carry):
        def row(r, carry2):
            ridx = jnp.full((16,), s_ * 80 + r, jnp.int32)
            v = plsc.load_gather(outbuf, [ridx])
            for j in range(h // 16):
                bbuf[r, pl.ds(j * 16, 16)] = v
            return carry2

        lax.fori_loop(0, 80, row, 0)
        base = sid * seg + s_ * 80

        @pl.when(cid == 0)
        def _():
            pltpu.sync_copy(bbuf, dgA.at[pl.ds(base, 80)])

        @pl.when(cid == 1)
        def _():
            pltpu.sync_copy(bbuf, dgB.at[pl.ds(base, 80)])

        return carry

    lax.fori_loop(0, seg // 80, slab, 0)


def _make_agg(n_nodes, h, n_chunks):
    f32 = jnp.float32
    outs = [jax.ShapeDtypeStruct((n_nodes, h), f32),
            jax.ShapeDtypeStruct((n_nodes, h), f32)]
    scratch = [
        pltpu.VMEM((_NIDX, 1, _CHUNK), jnp.int32),
        pltpu.VMEM((_NIDX, 1, _CHUNK), jnp.int32),
        pltpu.VMEM((_NBUF, _CHUNK, h), f32),
        pltpu.VMEM_SHARED((n_nodes, h), f32),
        pltpu.SemaphoreType.DMA((_NBUF,)),
        pltpu.SemaphoreType.DMA((_NIDX,)),
    ]
    mesh = plsc.VectorSubcoreMesh(core_axis_name="c", subcore_axis_name="s")
    return pl.kernel(
        functools.partial(_agg_body, n_nodes, n_chunks),
        out_type=tuple(outs), mesh=mesh, scratch_types=scratch,
    )


def _make_deg(n_pad, h, n_chunks):
    f32 = jnp.float32
    seg = n_pad // _NTILES
    outs = [jax.ShapeDtypeStruct((n_pad, h), f32),
            jax.ShapeDtypeStruct((n_pad, h), f32)]
    scratch = [
        pltpu.VMEM((n_chunks, _CHUNK), jnp.int32),
        pltpu.VMEM((1, n_pad), f32),
        pltpu.VMEM((_NTILES, seg), f32),
        pltpu.VMEM((seg,), f32),
        pltpu.VMEM((80, h), f32),
        pltpu.VMEM_SHARED((_NTILES, 1, n_pad), f32),
        pltpu.SemaphoreType.DMA,
    ]
    mesh = plsc.VectorSubcoreMesh(core_axis_name="c", subcore_axis_name="s")
    return pl.kernel(
        functools.partial(_deg_body, n_pad, n_chunks, h),
        out_type=tuple(outs), mesh=mesh, scratch_types=scratch,
        compiler_params=pltpu.CompilerParams(needs_layout_passes=False),
    )


def _l0_body(x_ref, ws_ref, wn_ref, s0_ref, t0_ref):
    xb = x_ref[...]
    s0_ref[...] = jnp.dot(xb, ws_ref[...], preferred_element_type=jnp.float32)
    t0_ref[...] = jnp.dot(xb, wn_ref[...], preferred_element_type=jnp.float32)


def _mid_body(s0_ref, pL_ref, pR_ref, dA_ref, dB_ref,
              wst_ref, wsb_ref, wnt_ref, wnb_ref,
              s1_ref, t1_ref):
    rs = jnp.maximum(s0_ref[...], 0.0)
    deg = dA_ref[:, 0:1] + dB_ref[:, 0:1]
    inv = 1.0 / jnp.maximum(deg, 1.0)
    a = pL_ref[...] + pR_ref[...]
    ra = jnp.maximum(a, 0.0) * inv
    s1_ref[...] = (jnp.dot(rs, wst_ref[...], preferred_element_type=jnp.float32)
                   + jnp.dot(ra, wsb_ref[...], preferred_element_type=jnp.float32))
    t1_ref[...] = (jnp.dot(rs, wnt_ref[...], preferred_element_type=jnp.float32)
                   + jnp.dot(ra, wnb_ref[...], preferred_element_type=jnp.float32))


def _out_body(s1_ref, pL_ref, pR_ref, dA_ref, dB_ref,
              wot_ref, wob_ref, o_ref):
    rs = jnp.maximum(s1_ref[...], 0.0)
    deg = dA_ref[:, 0:1] + dB_ref[:, 0:1]
    inv = 1.0 / jnp.maximum(deg, 1.0)
    a = pL_ref[...] + pR_ref[...]
    ra = jnp.maximum(a, 0.0) * inv
    o_ref[...] = (jnp.dot(rs, wot_ref[...], preferred_element_type=jnp.float32)
                  + jnp.dot(ra, wob_ref[...], preferred_element_type=jnp.float32))


def _row_spec(rb, w):
    return pl.BlockSpec((rb, w), lambda i: (i, 0))


def _full_spec(hh, w):
    return pl.BlockSpec((hh, w), lambda i: (0, 0))


def kernel(x, edge_index, W0_self, W0_neigh, W1_self, W1_neigh, W_out):
    f32 = jnp.float32
    n, d = x.shape
    e = edge_index.shape[1]
    h = W0_self.shape[1]
    c = W_out.shape[1]
    nw = _NSC * _NTILES
    assert e % (nw * _CHUNK) == 0 and n % _NTILES == 0
    n_chunks = e // (nw * _CHUNK)

    src4 = edge_index[0].reshape(nw, n_chunks, 1, _CHUNK)
    dst4 = edge_index[1].reshape(nw, n_chunks, 1, _CHUNK)
    dst3 = edge_index[1].reshape(nw, n_chunks, _CHUNK)
    zacc = jnp.zeros((n, h), f32)

    rb = 1000 if n % 1000 == 0 else n // 8
    grid = n // rb

    n_pad = -(-n // (_NTILES * 80)) * (_NTILES * 80)
    dgA, dgB = _make_deg(n_pad, h, n_chunks)(dst3)

    s0, t0 = pl.pallas_call(
        _l0_body,
        grid=(grid,),
        in_specs=[_row_spec(rb, d), _full_spec(d, h), _full_spec(d, h)],
        out_specs=[_row_spec(rb, h), _row_spec(rb, h)],
        out_shape=[jax.ShapeDtypeStruct((n, h), f32),
                   jax.ShapeDtypeStruct((n, h), f32)],
    )(x, W0_self, W0_neigh)

    p0L, p0R = _make_agg(n, h, n_chunks)(t0, src4, dst4, zacc)

    s1, t1 = pl.pallas_call(
        _mid_body,
        grid=(grid,),
        in_specs=[_row_spec(rb, h), _row_spec(rb, h), _row_spec(rb, h),
                  _row_spec(rb, h), _row_spec(rb, h),
                  _full_spec(h, h), _full_spec(h, h),
                  _full_spec(h, h), _full_spec(h, h)],
        out_specs=[_row_spec(rb, h), _row_spec(rb, h)],
        out_shape=[jax.ShapeDtypeStruct((n, h), f32),
                   jax.ShapeDtypeStruct((n, h), f32)],
    )(s0, p0L, p0R, dgA, dgB,
      W1_self[:h], W1_self[h:], W1_neigh[:h], W1_neigh[h:])

    p1L, p1R = _make_agg(n, h, n_chunks)(t1, src4, dst4, zacc)

    out = pl.pallas_call(
        _out_body,
        grid=(grid,),
        in_specs=[_row_spec(rb, h), _row_spec(rb, h), _row_spec(rb, h),
                  _row_spec(rb, h), _row_spec(rb, h),
                  _full_spec(h, c), _full_spec(h, c)],
        out_specs=_row_spec(rb, c),
        out_shape=jax.ShapeDtypeStruct((n, c), f32),
    )(s1, p1L, p1R, dgA, dgB, W_out[:h], W_out[h:])

    return out


# speedup vs baseline: 1.0847x; 1.0616x over previous
import functools

import jax
import jax.numpy as jnp
from jax import lax
from jax.experimental import pallas as pl
from jax.experimental.pallas import tpu as pltpu
from jax.experimental.pallas import tpu_sc as plsc

from math import gcd as _gcd

_CHUNK = 80
_NTILES = 16
_NSC = 2
_NBUF = 3
_NIDX = 6


def _ranged_copy(src_ref, dst_ref, r0, rpt, tail0, rem, is_last):
    pltpu.sync_copy(src_ref.at[pl.ds(r0, rpt)], dst_ref.at[pl.ds(r0, rpt)])
    if rem:
        @pl.when(is_last)
        def _():
            pltpu.sync_copy(src_ref.at[pl.ds(tail0, rem)],
                            dst_ref.at[pl.ds(tail0, rem)])


def _agg_body(n_nodes, n_chunks,
              t, src, dst,
              pL, pR, sslot, dslot, rows_v, acc_sh, gsem, isem):
    cid = lax.axis_index("c")
    sid = lax.axis_index("s")
    wid = cid * _NTILES + sid
    rpt = (n_nodes // _NTILES) // 8 * 8
    tail0 = _NTILES * rpt
    rem = n_nodes - tail0
    r0 = sid * rpt
    is_last = sid == _NTILES - 1
    ebase = wid * n_chunks * _CHUNK

    zero16 = jnp.zeros((16,), jnp.float32)

    def zrow(i, carry):
        r = i // 8
        rows_v[0, r, pl.ds((i % 8) * 16, 16)] = zero16
        return carry

    lax.fori_loop(0, _CHUNK * 8, zrow, 0)
    for q in range(8):
        pltpu.sync_copy(rows_v.at[0],
                        acc_sh.at[pl.ds(r0 + q * _CHUNK, _CHUNK)])
    plsc.subcore_barrier()

    def load_idx(cc, j):
        off = ebase + cc * _CHUNK
        pltpu.async_copy(src.at[pl.ds(off, _CHUNK)], sslot.at[j], isem.at[j])
        pltpu.async_copy(dst.at[pl.ds(off, _CHUNK)], dslot.at[j], isem.at[j])

    def wait_idx(cc, j):
        off = ebase + cc * _CHUNK
        pltpu.make_async_copy(src.at[pl.ds(off, _CHUNK)], sslot.at[j],
                              isem.at[j]).wait()
        pltpu.make_async_copy(dst.at[pl.ds(off, _CHUNK)], dslot.at[j],
                              isem.at[j]).wait()

    for j in range(_NIDX):
        load_idx(j, j)
    for b in range(_NBUF):
        wait_idx(b, b)
        pltpu.async_copy(t.at[sslot.at[b]], rows_v.at[b], gsem.at[b])

    steps_per_group = _NIDX * _NBUF // _gcd(_NIDX, _NBUF)
    n_groups = -(-n_chunks // steps_per_group)

    def ring(g, carry):
        for k in range(steps_per_group):
            c = g * steps_per_group + k
            b = k % _NBUF
            j = k % _NIDX

            @pl.when(c < n_chunks)
            def _():
                pltpu.make_async_copy(t.at[sslot.at[j]], rows_v.at[b],
                                      gsem.at[b]).wait()
                pltpu.sync_copy(rows_v.at[b], acc_sh.at[dslot.at[j]],
                                add=True)

            @pl.when(c + _NBUF < n_chunks)
            def _():
                jn = (k + _NBUF) % _NIDX
                wait_idx(c + _NBUF, jn)
                pltpu.async_copy(t.at[sslot.at[jn]], rows_v.at[b],
                                 gsem.at[b])

            @pl.when(c + _NIDX < n_chunks)
            def _():
                load_idx(c + _NIDX, j)
        return carry

    lax.fori_loop(0, n_groups, ring, 0)
    plsc.subcore_barrier()

    @pl.when(cid == 0)
    def _():
        _ranged_copy(acc_sh, pL, r0, rpt, tail0, rem, is_last)

    @pl.when(cid == 1)
    def _():
        _ranged_copy(acc_sh, pR, r0, rpt, tail0, rem, is_last)


def _deg_body(n_pad, n_chunks, h,
              dst, dgA, dgB,
              dst_v, hist_v, sbuf, outbuf, bbuf, deg_sh, sem):
    cid = lax.axis_index("c")
    sid = lax.axis_index("s")
    wid = cid * _NTILES + sid
    seg = n_pad // _NTILES
    f32 = jnp.float32
    zero16 = jnp.zeros((16,), f32)
    izero16 = jnp.zeros((16,), jnp.int32)
    ones16 = jnp.ones((16,), f32)
    n_edges_tile = n_chunks * _CHUNK

    pltpu.sync_copy(dst.at[pl.ds(wid * n_edges_tile, n_edges_tile)], dst_v)

    def zhist(i, carry):
        hist_v[0, pl.ds(i * 16, 16)] = zero16
        return carry

    lax.fori_loop(0, n_pad // 16, zhist, 0)

    def count(c, carry):
        for k in range(_CHUNK // 16):
            iv = dst_v[pl.ds(c * _CHUNK + k * 16, 16)]
            plsc.addupdate_scatter(hist_v, [izero16, iv], ones16)
        return carry

    lax.fori_loop(0, n_chunks, count, 0)

    pltpu.sync_copy(hist_v, deg_sh.at[sid])
    plsc.subcore_barrier()

    for r in range(_NTILES):
        pltpu.sync_copy(deg_sh.at[r, 0, pl.ds(sid * seg, seg)], sbuf.at[r])

    def comb(i, carry):
        s = sbuf[0, pl.ds(i * 16, 16)]
        for r in range(1, _NTILES):
            s = s + sbuf[r, pl.ds(i * 16, 16)]
        outbuf[pl.ds(i * 16, 16)] = s
        return carry

    lax.fori_loop(0, seg // 16, comb, 0)

    def slab(s_, carry):
        def row(r, carry2):
            ridx = jnp.full((16,), s_ * 80 + r, jnp.int32)
            v = plsc.load_gather(outbuf, [ridx])
            for j in range(h // 16):
                bbuf[r, pl.ds(j * 16, 16)] = v
            return carry2

        lax.fori_loop(0, 80, row, 0)
        base = sid * seg + s_ * 80

        @pl.when(cid == 0)
        def _():
            pltpu.sync_copy(bbuf, dgA.at[pl.ds(base, 80)])

        @pl.when(cid == 1)
        def _():
            pltpu.sync_copy(bbuf, dgB.at[pl.ds(base, 80)])

        return carry

    lax.fori_loop(0, seg // 80, slab, 0)


def _make_agg(n_nodes, h, n_chunks):
    f32 = jnp.float32
    outs = [jax.ShapeDtypeStruct((n_nodes, h), f32),
            jax.ShapeDtypeStruct((n_nodes, h), f32)]
    scratch = [
        pltpu.VMEM((_NIDX, _CHUNK), jnp.int32),
        pltpu.VMEM((_NIDX, _CHUNK), jnp.int32),
        pltpu.VMEM((_NBUF, _CHUNK, h), f32),
        pltpu.VMEM_SHARED((n_nodes, h), f32),
        pltpu.SemaphoreType.DMA((_NBUF,)),
        pltpu.SemaphoreType.DMA((_NIDX,)),
    ]
    mesh = plsc.VectorSubcoreMesh(core_axis_name="c", subcore_axis_name="s")
    return pl.kernel(
        functools.partial(_agg_body, n_nodes, n_chunks),
        out_type=tuple(outs), mesh=mesh, scratch_types=scratch,
    )


def _make_deg(n_pad, h, n_chunks):
    f32 = jnp.float32
    seg = n_pad // _NTILES
    outs = [jax.ShapeDtypeStruct((n_pad, h), f32),
            jax.ShapeDtypeStruct((n_pad, h), f32)]
    scratch = [
        pltpu.VMEM((n_chunks * _CHUNK,), jnp.int32),
        pltpu.VMEM((1, n_pad), f32),
        pltpu.VMEM((_NTILES, seg), f32),
        pltpu.VMEM((seg,), f32),
        pltpu.VMEM((80, h), f32),
        pltpu.VMEM_SHARED((_NTILES, 1, n_pad), f32),
        pltpu.SemaphoreType.DMA,
    ]
    mesh = plsc.VectorSubcoreMesh(core_axis_name="c", subcore_axis_name="s")
    return pl.kernel(
        functools.partial(_deg_body, n_pad, n_chunks, h),
        out_type=tuple(outs), mesh=mesh, scratch_types=scratch,
        compiler_params=pltpu.CompilerParams(needs_layout_passes=False),
    )


def _l0_body(x_ref, ws_ref, wn_ref, s0_ref, t0_ref):
    xb = x_ref[...]
    s0_ref[...] = jnp.dot(xb, ws_ref[...], preferred_element_type=jnp.float32)
    t0_ref[...] = jnp.dot(xb, wn_ref[...], preferred_element_type=jnp.float32)


def _mid_body(s0_ref, pL_ref, pR_ref, dA_ref, dB_ref,
              wst_ref, wsb_ref, wnt_ref, wnb_ref,
              s1_ref, t1_ref):
    rs = jnp.maximum(s0_ref[...], 0.0)
    deg = (dA_ref[:, 0:1].astype(jnp.float32)
           + dB_ref[:, 0:1].astype(jnp.float32))
    inv = 1.0 / jnp.maximum(deg, 1.0)
    a = pL_ref[...] + pR_ref[...]
    ra = jnp.maximum(a, 0.0) * inv
    s1_ref[...] = (jnp.dot(rs, wst_ref[...], preferred_element_type=jnp.float32)
                   + jnp.dot(ra, wsb_ref[...], preferred_element_type=jnp.float32))
    t1_ref[...] = (jnp.dot(rs, wnt_ref[...], preferred_element_type=jnp.float32)
                   + jnp.dot(ra, wnb_ref[...], preferred_element_type=jnp.float32))


def _out_body(s1_ref, pL_ref, pR_ref, dA_ref, dB_ref,
              wot_ref, wob_ref, o_ref):
    rs = jnp.maximum(s1_ref[...], 0.0)
    deg = (dA_ref[:, 0:1].astype(jnp.float32)
           + dB_ref[:, 0:1].astype(jnp.float32))
    inv = 1.0 / jnp.maximum(deg, 1.0)
    a = pL_ref[...] + pR_ref[...]
    ra = jnp.maximum(a, 0.0) * inv
    o_ref[...] = (jnp.dot(rs, wot_ref[...], preferred_element_type=jnp.float32)
                  + jnp.dot(ra, wob_ref[...], preferred_element_type=jnp.float32))


def _row_spec(rb, w):
    return pl.BlockSpec((rb, w), lambda i: (i, 0))


def _full_spec(hh, w):
    return pl.BlockSpec((hh, w), lambda i: (0, 0))


def kernel(x, edge_index, W0_self, W0_neigh, W1_self, W1_neigh, W_out):
    f32 = jnp.float32
    n, d = x.shape
    e = edge_index.shape[1]
    h = W0_self.shape[1]
    c = W_out.shape[1]
    nw = _NSC * _NTILES
    assert e % (nw * _CHUNK) == 0 and n % _NTILES == 0
    n_chunks = e // (nw * _CHUNK)

    src1 = edge_index[0]
    dst1 = edge_index[1]

    rb = 1000 if n % 1000 == 0 else n // 8
    grid = n // rb

    n_pad = -(-n // (_NTILES * 80)) * (_NTILES * 80)
    dgA, dgB = _make_deg(n_pad, h, n_chunks)(dst1)

    s0, t0 = pl.pallas_call(
        _l0_body,
        grid=(grid,),
        in_specs=[_row_spec(rb, d), _full_spec(d, h), _full_spec(d, h)],
        out_specs=[_row_spec(rb, h), _row_spec(rb, h)],
        out_shape=[jax.ShapeDtypeStruct((n, h), f32),
                   jax.ShapeDtypeStruct((n, h), f32)],
    )(x, W0_self, W0_neigh)

    p0L, p0R = _make_agg(n, h, n_chunks)(t0, src1, dst1)

    s1, t1 = pl.pallas_call(
        _mid_body,
        grid=(grid,),
        in_specs=[_row_spec(rb, h), _row_spec(rb, h), _row_spec(rb, h),
                  _row_spec(rb, h), _row_spec(rb, h),
                  _full_spec(h, h), _full_spec(h, h),
                  _full_spec(h, h), _full_spec(h, h)],
        out_specs=[_row_spec(rb, h), _row_spec(rb, h)],
        out_shape=[jax.ShapeDtypeStruct((n, h), f32),
                   jax.ShapeDtypeStruct((n, h), f32)],
    )(s0, p0L, p0R, dgA, dgB,
      W1_self[:h], W1_self[h:], W1_neigh[:h], W1_neigh[h:])

    p1L, p1R = _make_agg(n, h, n_chunks)(t1, src1, dst1)

    out = pl.pallas_call(
        _out_body,
        grid=(grid,),
        in_specs=[_row_spec(rb, h), _row_spec(rb, h), _row_spec(rb, h),
                  _row_spec(rb, h), _row_spec(rb, h),
                  _full_spec(h, c), _full_spec(h, c)],
        out_specs=_row_spec(rb, c),
        out_shape=jax.ShapeDtypeStruct((n, c), f32),
    )(s1, p1L, p1R, dgA, dgB, W_out[:h], W_out[h:])

    return out


# speedup vs baseline: 1.1156x; 1.0286x over previous
import functools

import jax
import jax.numpy as jnp
from jax import lax
from jax.experimental import pallas as pl
from jax.experimental.pallas import tpu as pltpu
from jax.experimental.pallas import tpu_sc as plsc

from math import gcd as _gcd

_CHUNK = 80
_NTILES = 16
_NSC = 2
_NBUF = 3
_NIDX = 6


def _ranged_copy(src_ref, dst_ref, r0, rpt, tail0, rem, is_last):
    pltpu.sync_copy(src_ref.at[pl.ds(r0, rpt)], dst_ref.at[pl.ds(r0, rpt)])
    if rem:
        @pl.when(is_last)
        def _():
            pltpu.sync_copy(src_ref.at[pl.ds(tail0, rem)],
                            dst_ref.at[pl.ds(tail0, rem)])


def _agg_body(n_nodes, n_chunks,
              t, src, dst,
              pL, pR, sslot, dslot, rows_v, acc_sh, gsem, isem):
    cid = lax.axis_index("c")
    sid = lax.axis_index("s")
    wid = cid * _NTILES + sid
    rpt = (n_nodes // _NTILES) // 8 * 8
    tail0 = _NTILES * rpt
    rem = n_nodes - tail0
    r0 = sid * rpt
    is_last = sid == _NTILES - 1
    ebase = wid * n_chunks * _CHUNK

    zero16 = jnp.zeros((16,), jnp.float32)

    def zrow(i, carry):
        r = i // 8
        rows_v[0, r, pl.ds((i % 8) * 16, 16)] = zero16
        return carry

    lax.fori_loop(0, _CHUNK * 8, zrow, 0)
    for q in range(8):
        pltpu.sync_copy(rows_v.at[0],
                        acc_sh.at[pl.ds(r0 + q * _CHUNK, _CHUNK)])
    plsc.subcore_barrier()

    def load_idx(cc, j):
        off = ebase + cc * _CHUNK
        pltpu.async_copy(src.at[pl.ds(off, _CHUNK)], sslot.at[j], isem.at[j])
        pltpu.async_copy(dst.at[pl.ds(off, _CHUNK)], dslot.at[j], isem.at[j])

    def wait_idx(cc, j):
        off = ebase + cc * _CHUNK
        pltpu.make_async_copy(src.at[pl.ds(off, _CHUNK)], sslot.at[j],
                              isem.at[j]).wait()
        pltpu.make_async_copy(dst.at[pl.ds(off, _CHUNK)], dslot.at[j],
                              isem.at[j]).wait()

    for j in range(_NIDX):
        load_idx(j, j)
    for b in range(_NBUF):
        wait_idx(b, b)
        pltpu.async_copy(t.at[sslot.at[b]], rows_v.at[b], gsem.at[b])

    steps_per_group = _NIDX * _NBUF // _gcd(_NIDX, _NBUF)
    n_groups = -(-n_chunks // steps_per_group)

    def ring(g, carry):
        for k in range(steps_per_group):
            c = g * steps_per_group + k
            b = k % _NBUF
            j = k % _NIDX

            @pl.when(c < n_chunks)
            def _():
                pltpu.make_async_copy(t.at[sslot.at[j]], rows_v.at[b],
                                      gsem.at[b]).wait()
                pltpu.sync_copy(rows_v.at[b], acc_sh.at[dslot.at[j]],
                                add=True)

            @pl.when(c + _NBUF < n_chunks)
            def _():
                jn = (k + _NBUF) % _NIDX
                wait_idx(c + _NBUF, jn)
                pltpu.async_copy(t.at[sslot.at[jn]], rows_v.at[b],
                                 gsem.at[b])

            @pl.when(c + _NIDX < n_chunks)
            def _():
                load_idx(c + _NIDX, j)
        return carry

    lax.fori_loop(0, n_groups, ring, 0)
    plsc.subcore_barrier()

    @pl.when(cid == 0)
    def _():
        _ranged_copy(acc_sh, pL, r0, rpt, tail0, rem, is_last)

    @pl.when(cid == 1)
    def _():
        _ranged_copy(acc_sh, pR, r0, rpt, tail0, rem, is_last)


def _deg_body(n_pad, n_chunks, h,
              dst, dgA, dgB,
              dst_v, hist_v, sbuf, outbuf, bbuf, deg_sh, sem):
    cid = lax.axis_index("c")
    sid = lax.axis_index("s")
    wid = cid * _NTILES + sid
    seg = n_pad // _NTILES
    f32 = jnp.float32
    zero16 = jnp.zeros((16,), f32)
    izero16 = jnp.zeros((16,), jnp.int32)
    ones16 = jnp.ones((16,), f32)
    n_edges_tile = n_chunks * _CHUNK

    pltpu.sync_copy(dst.at[pl.ds(wid * n_edges_tile, n_edges_tile)], dst_v)

    def zhist(i, carry):
        hist_v[0, pl.ds(i * 16, 16)] = zero16
        return carry

    lax.fori_loop(0, n_pad // 16, zhist, 0)

    def count(c, carry):
        for k in range(_CHUNK // 16):
            iv = dst_v[pl.ds(c * _CHUNK + k * 16, 16)]
            plsc.addupdate_scatter(hist_v, [izero16, iv], ones16)
        return carry

    lax.fori_loop(0, n_chunks, count, 0)

    pltpu.sync_copy(hist_v, deg_sh.at[sid])
    plsc.subcore_barrier()

    for r in range(_NTILES):
        pltpu.sync_copy(deg_sh.at[r, 0, pl.ds(sid * seg, seg)], sbuf.at[r])

    def comb(i, carry):
        s = sbuf[0, pl.ds(i * 16, 16)]
        for r in range(1, _NTILES):
            s = s + sbuf[r, pl.ds(i * 16, 16)]
        outbuf[pl.ds(i * 16, 16)] = s
        return carry

    lax.fori_loop(0, seg // 16, comb, 0)

    def slab(s_, carry):
        def row(r, carry2):
            ridx = jnp.full((16,), s_ * 80 + r, jnp.int32)
            v = plsc.load_gather(outbuf, [ridx])
            for j in range(h // 16):
                bbuf[r, pl.ds(j * 16, 16)] = v
            return carry2

        lax.fori_loop(0, 80, row, 0)
        base = sid * seg + s_ * 80

        @pl.when(cid == 0)
        def _():
            pltpu.sync_copy(bbuf, dgA.at[pl.ds(base, 80)])

        @pl.when(cid == 1)
        def _():
            pltpu.sync_copy(bbuf, dgB.at[pl.ds(base, 80)])

        return carry

    lax.fori_loop(0, seg // 80, slab, 0)


def _make_agg(n_nodes, h, n_chunks):
    f32 = jnp.float32
    outs = [jax.ShapeDtypeStruct((n_nodes, h), f32),
            jax.ShapeDtypeStruct((n_nodes, h), f32)]
    scratch = [
        pltpu.VMEM((_NIDX, _CHUNK), jnp.int32),
        pltpu.VMEM((_NIDX, _CHUNK), jnp.int32),
        pltpu.VMEM((_NBUF, _CHUNK, h), f32),
        pltpu.VMEM_SHARED((n_nodes, h), f32),
        pltpu.SemaphoreType.DMA((_NBUF,)),
        pltpu.SemaphoreType.DMA((_NIDX,)),
    ]
    mesh = plsc.VectorSubcoreMesh(core_axis_name="c", subcore_axis_name="s")
    return pl.kernel(
        functools.partial(_agg_body, n_nodes, n_chunks),
        out_type=tuple(outs), mesh=mesh, scratch_types=scratch,
    )


def _make_deg(n_pad, h, n_chunks):
    f32 = jnp.float32
    seg = n_pad // _NTILES
    outs = [jax.ShapeDtypeStruct((n_pad, h), f32),
            jax.ShapeDtypeStruct((n_pad, h), f32)]
    scratch = [
        pltpu.VMEM((n_chunks * _CHUNK,), jnp.int32),
        pltpu.VMEM((1, n_pad), f32),
        pltpu.VMEM((_NTILES, seg), f32),
        pltpu.VMEM((seg,), f32),
        pltpu.VMEM((80, h), f32),
        pltpu.VMEM_SHARED((_NTILES, 1, n_pad), f32),
        pltpu.SemaphoreType.DMA,
    ]
    mesh = plsc.VectorSubcoreMesh(core_axis_name="c", subcore_axis_name="s")
    return pl.kernel(
        functools.partial(_deg_body, n_pad, n_chunks, h),
        out_type=tuple(outs), mesh=mesh, scratch_types=scratch,
        compiler_params=pltpu.CompilerParams(needs_layout_passes=False),
    )


def _l0_body(x_ref, ws_ref, wn_ref, s0_ref, t0_ref):
    xb = x_ref[...]
    s0_ref[...] = jnp.dot(xb, ws_ref[...], preferred_element_type=jnp.float32)
    t0_ref[...] = jnp.dot(xb, wn_ref[...], preferred_element_type=jnp.float32)


def _mid_body(s0_ref, pL_ref, pR_ref, dA_ref, dB_ref,
              wst_ref, wsb_ref, wnt_ref, wnb_ref,
              s1_ref, t1_ref):
    rs = jnp.maximum(s0_ref[...], 0.0)
    deg = (dA_ref[:, 0:1].astype(jnp.float32)
           + dB_ref[:, 0:1].astype(jnp.float32))
    inv = 1.0 / jnp.maximum(deg, 1.0)
    a = pL_ref[...] + pR_ref[...]
    ra = jnp.maximum(a, 0.0) * inv
    s1_ref[...] = (jnp.dot(rs, wst_ref[...], preferred_element_type=jnp.float32)
                   + jnp.dot(ra, wsb_ref[...], preferred_element_type=jnp.float32))
    t1_ref[...] = (jnp.dot(rs, wnt_ref[...], preferred_element_type=jnp.float32)
                   + jnp.dot(ra, wnb_ref[...], preferred_element_type=jnp.float32))


def _out_body(s1_ref, pL_ref, pR_ref, dA_ref, dB_ref,
              wot_ref, wob_ref, o_ref):
    rs = jnp.maximum(s1_ref[...], 0.0)
    deg = (dA_ref[:, 0:1].astype(jnp.float32)
           + dB_ref[:, 0:1].astype(jnp.float32))
    inv = 1.0 / jnp.maximum(deg, 1.0)
    a = pL_ref[...] + pR_ref[...]
    ra = jnp.maximum(a, 0.0) * inv
    o_ref[...] = (jnp.dot(rs, wot_ref[...], preferred_element_type=jnp.float32)
                  + jnp.dot(ra, wob_ref[...], preferred_element_type=jnp.float32))


def _row_spec(rb, w):
    return pl.BlockSpec((rb, w), lambda i: (i, 0))


def _full_spec(hh, w):
    return pl.BlockSpec((hh, w), lambda i: (0, 0))


def kernel(x, edge_index, W0_self, W0_neigh, W1_self, W1_neigh, W_out):
    f32 = jnp.float32
    n, d = x.shape
    e = edge_index.shape[1]
    h = W0_self.shape[1]
    c = W_out.shape[1]
    nw = _NSC * _NTILES
    assert e % (nw * _CHUNK) == 0 and n % _NTILES == 0
    n_chunks = e // (nw * _CHUNK)

    src1 = edge_index[0]
    dst1 = edge_index[1]

    rb = 2000 if n % 2000 == 0 else n // 8
    grid = n // rb

    n_pad = -(-n // (_NTILES * 80)) * (_NTILES * 80)
    dgA, dgB = _make_deg(n_pad, h, n_chunks)(dst1)

    s0, t0 = pl.pallas_call(
        _l0_body,
        grid=(grid,),
        in_specs=[_row_spec(rb, d), _full_spec(d, h), _full_spec(d, h)],
        out_specs=[_row_spec(rb, h), _row_spec(rb, h)],
        out_shape=[jax.ShapeDtypeStruct((n, h), f32),
                   jax.ShapeDtypeStruct((n, h), f32)],
    )(x, W0_self, W0_neigh)

    p0L, p0R = _make_agg(n, h, n_chunks)(t0, src1, dst1)

    s1, t1 = pl.pallas_call(
        _mid_body,
        grid=(grid,),
        in_specs=[_row_spec(rb, h), _row_spec(rb, h), _row_spec(rb, h),
                  _row_spec(rb, h), _row_spec(rb, h),
                  _full_spec(h, h), _full_spec(h, h),
                  _full_spec(h, h), _full_spec(h, h)],
        out_specs=[_row_spec(rb, h), _row_spec(rb, h)],
        out_shape=[jax.ShapeDtypeStruct((n, h), f32),
                   jax.ShapeDtypeStruct((n, h), f32)],
    )(s0, p0L, p0R, dgA, dgB,
      W1_self[:h], W1_self[h:], W1_neigh[:h], W1_neigh[h:])

    p1L, p1R = _make_agg(n, h, n_chunks)(t1, src1, dst1)

    out = pl.pallas_call(
        _out_body,
        grid=(grid,),
        in_specs=[_row_spec(rb, h), _row_spec(rb, h), _row_spec(rb, h),
                  _row_spec(rb, h), _row_spec(rb, h),
                  _full_spec(h, c), _full_spec(h, c)],
        out_specs=_row_spec(rb, c),
        out_shape=jax.ShapeDtypeStruct((n, c), f32),
    )(s1, p1L, p1R, dgA, dgB, W_out[:h], W_out[h:])

    return out


# speedup vs baseline: 1.1173x; 1.0015x over previous
import functools

import jax
import jax.numpy as jnp
from jax import lax
from jax.experimental import pallas as pl
from jax.experimental.pallas import tpu as pltpu
from jax.experimental.pallas import tpu_sc as plsc

from math import gcd as _gcd

_CHUNK = 80
_NTILES = 16
_NSC = 2
_NBUF = 3
_NIDX = 6


def _ranged_copy(src_ref, dst_ref, r0, rpt, tail0, rem, is_last):
    pltpu.sync_copy(src_ref.at[pl.ds(r0, rpt)], dst_ref.at[pl.ds(r0, rpt)])
    if rem:
        @pl.when(is_last)
        def _():
            pltpu.sync_copy(src_ref.at[pl.ds(tail0, rem)],
                            dst_ref.at[pl.ds(tail0, rem)])


def _agg_body(n_nodes, n_chunks,
              t, src, dst,
              pL, pR, sslot, dslot, rows_v, acc_sh, gsem, isem):
    cid = lax.axis_index("c")
    sid = lax.axis_index("s")
    wid = cid * _NTILES + sid
    rpt = (n_nodes // _NTILES) // 8 * 8
    tail0 = _NTILES * rpt
    rem = n_nodes - tail0
    r0 = sid * rpt
    is_last = sid == _NTILES - 1
    ebase = wid * n_chunks * _CHUNK

    zero16 = jnp.zeros((16,), jnp.float32)

    def zrow(i, carry):
        r = i // 8
        rows_v[0, r, pl.ds((i % 8) * 16, 16)] = zero16
        return carry

    lax.fori_loop(0, _CHUNK * 8, zrow, 0)
    for q in range(8):
        pltpu.sync_copy(rows_v.at[0],
                        acc_sh.at[pl.ds(r0 + q * _CHUNK, _CHUNK)])
    plsc.subcore_barrier()

    def load_idx(cc, j):
        off = ebase + cc * _CHUNK
        pltpu.async_copy(src.at[pl.ds(off, _CHUNK)], sslot.at[j], isem.at[j])
        pltpu.async_copy(dst.at[pl.ds(off, _CHUNK)], dslot.at[j], isem.at[j])

    def wait_idx(cc, j):
        off = ebase + cc * _CHUNK
        pltpu.make_async_copy(src.at[pl.ds(off, _CHUNK)], sslot.at[j],
                              isem.at[j]).wait()
        pltpu.make_async_copy(dst.at[pl.ds(off, _CHUNK)], dslot.at[j],
                              isem.at[j]).wait()

    for j in range(_NIDX):
        load_idx(j, j)
    for b in range(_NBUF):
        wait_idx(b, b)
        pltpu.async_copy(t.at[sslot.at[b]], rows_v.at[b], gsem.at[b],
                         priority=b % 2)

    steps_per_group = _NIDX * _NBUF // _gcd(_NIDX, _NBUF)
    n_groups = -(-n_chunks // steps_per_group)

    def ring(g, carry):
        for k in range(steps_per_group):
            c = g * steps_per_group + k
            b = k % _NBUF
            j = k % _NIDX

            @pl.when(c < n_chunks)
            def _():
                pltpu.make_async_copy(t.at[sslot.at[j]], rows_v.at[b],
                                      gsem.at[b]).wait()
                pltpu.sync_copy(rows_v.at[b], acc_sh.at[dslot.at[j]],
                                add=True)

            @pl.when(c + _NBUF < n_chunks)
            def _():
                jn = (k + _NBUF) % _NIDX
                wait_idx(c + _NBUF, jn)
                pltpu.async_copy(t.at[sslot.at[jn]], rows_v.at[b],
                                 gsem.at[b], priority=b % 2)

            @pl.when(c + _NIDX < n_chunks)
            def _():
                load_idx(c + _NIDX, j)
        return carry

    lax.fori_loop(0, n_groups, ring, 0)
    plsc.subcore_barrier()

    @pl.when(cid == 0)
    def _():
        _ranged_copy(acc_sh, pL, r0, rpt, tail0, rem, is_last)

    @pl.when(cid == 1)
    def _():
        _ranged_copy(acc_sh, pR, r0, rpt, tail0, rem, is_last)


def _deg_body(n_pad, n_chunks, h,
              dst, dgA, dgB,
              dst_v, hist_v, sbuf, outbuf, bbuf, deg_sh, sem):
    cid = lax.axis_index("c")
    sid = lax.axis_index("s")
    wid = cid * _NTILES + sid
    seg = n_pad // _NTILES
    f32 = jnp.float32
    zero16 = jnp.zeros((16,), f32)
    izero16 = jnp.zeros((16,), jnp.int32)
    ones16 = jnp.ones((16,), f32)
    n_edges_tile = n_chunks * _CHUNK

    pltpu.sync_copy(dst.at[pl.ds(wid * n_edges_tile, n_edges_tile)], dst_v)

    def zhist(i, carry):
        hist_v[0, pl.ds(i * 16, 16)] = zero16
        return carry

    lax.fori_loop(0, n_pad // 16, zhist, 0)

    def count(c, carry):
        for k in range(_CHUNK // 16):
            iv = dst_v[pl.ds(c * _CHUNK + k * 16, 16)]
            plsc.addupdate_scatter(hist_v, [izero16, iv], ones16)
        return carry

    lax.fori_loop(0, n_chunks, count, 0)

    pltpu.sync_copy(hist_v, deg_sh.at[sid])
    plsc.subcore_barrier()

    for r in range(_NTILES):
        pltpu.sync_copy(deg_sh.at[r, 0, pl.ds(sid * seg, seg)], sbuf.at[r])

    def comb(i, carry):
        s = sbuf[0, pl.ds(i * 16, 16)]
        for r in range(1, _NTILES):
            s = s + sbuf[r, pl.ds(i * 16, 16)]
        outbuf[pl.ds(i * 16, 16)] = s
        return carry

    lax.fori_loop(0, seg // 16, comb, 0)

    def slab(s_, carry):
        def row(r, carry2):
            ridx = jnp.full((16,), s_ * 80 + r, jnp.int32)
            v = plsc.load_gather(outbuf, [ridx])
            for j in range(h // 16):
                bbuf[r, pl.ds(j * 16, 16)] = v
            return carry2

        lax.fori_loop(0, 80, row, 0)
        base = sid * seg + s_ * 80

        @pl.when(cid == 0)
        def _():
            pltpu.sync_copy(bbuf, dgA.at[pl.ds(base, 80)])

        @pl.when(cid == 1)
        def _():
            pltpu.sync_copy(bbuf, dgB.at[pl.ds(base, 80)])

        return carry

    lax.fori_loop(0, seg // 80, slab, 0)


def _make_agg(n_nodes, h, n_chunks):
    f32 = jnp.float32
    outs = [jax.ShapeDtypeStruct((n_nodes, h), f32),
            jax.ShapeDtypeStruct((n_nodes, h), f32)]
    scratch = [
        pltpu.VMEM((_NIDX, _CHUNK), jnp.int32),
        pltpu.VMEM((_NIDX, _CHUNK), jnp.int32),
        pltpu.VMEM((_NBUF, _CHUNK, h), f32),
        pltpu.VMEM_SHARED((n_nodes, h), f32),
        pltpu.SemaphoreType.DMA((_NBUF,)),
        pltpu.SemaphoreType.DMA((_NIDX,)),
    ]
    mesh = plsc.VectorSubcoreMesh(core_axis_name="c", subcore_axis_name="s")
    return pl.kernel(
        functools.partial(_agg_body, n_nodes, n_chunks),
        out_type=tuple(outs), mesh=mesh, scratch_types=scratch,
    )


def _make_deg(n_pad, h, n_chunks):
    f32 = jnp.float32
    seg = n_pad // _NTILES
    outs = [jax.ShapeDtypeStruct((n_pad, h), f32),
            jax.ShapeDtypeStruct((n_pad, h), f32)]
    scratch = [
        pltpu.VMEM((n_chunks * _CHUNK,), jnp.int32),
        pltpu.VMEM((1, n_pad), f32),
        pltpu.VMEM((_NTILES, seg), f32),
        pltpu.VMEM((seg,), f32),
        pltpu.VMEM((80, h), f32),
        pltpu.VMEM_SHARED((_NTILES, 1, n_pad), f32),
        pltpu.SemaphoreType.DMA,
    ]
    mesh = plsc.VectorSubcoreMesh(core_axis_name="c", subcore_axis_name="s")
    return pl.kernel(
        functools.partial(_deg_body, n_pad, n_chunks, h),
        out_type=tuple(outs), mesh=mesh, scratch_types=scratch,
        compiler_params=pltpu.CompilerParams(needs_layout_passes=False),
    )


def _l0_body(x_ref, ws_ref, wn_ref, s0_ref, t0_ref):
    xb = x_ref[...]
    s0_ref[...] = jnp.dot(xb, ws_ref[...], preferred_element_type=jnp.float32)
    t0_ref[...] = jnp.dot(xb, wn_ref[...], preferred_element_type=jnp.float32)


def _mid_body(s0_ref, pL_ref, pR_ref, dA_ref, dB_ref,
              wst_ref, wsb_ref, wnt_ref, wnb_ref,
              s1_ref, t1_ref):
    rs = jnp.maximum(s0_ref[...], 0.0)
    deg = (dA_ref[:, 0:1].astype(jnp.float32)
           + dB_ref[:, 0:1].astype(jnp.float32))
    inv = 1.0 / jnp.maximum(deg, 1.0)
    a = pL_ref[...] + pR_ref[...]
    ra = jnp.maximum(a, 0.0) * inv
    s1_ref[...] = (jnp.dot(rs, wst_ref[...], preferred_element_type=jnp.float32)
                   + jnp.dot(ra, wsb_ref[...], preferred_element_type=jnp.float32))
    t1_ref[...] = (jnp.dot(rs, wnt_ref[...], preferred_element_type=jnp.float32)
                   + jnp.dot(ra, wnb_ref[...], preferred_element_type=jnp.float32))


def _out_body(s1_ref, pL_ref, pR_ref, dA_ref, dB_ref,
              wot_ref, wob_ref, o_ref):
    rs = jnp.maximum(s1_ref[...], 0.0)
    deg = (dA_ref[:, 0:1].astype(jnp.float32)
           + dB_ref[:, 0:1].astype(jnp.float32))
    inv = 1.0 / jnp.maximum(deg, 1.0)
    a = pL_ref[...] + pR_ref[...]
    ra = jnp.maximum(a, 0.0) * inv
    o_ref[...] = (jnp.dot(rs, wot_ref[...], preferred_element_type=jnp.float32)
                  + jnp.dot(ra, wob_ref[...], preferred_element_type=jnp.float32))


def _row_spec(rb, w):
    return pl.BlockSpec((rb, w), lambda i: (i, 0))


def _full_spec(hh, w):
    return pl.BlockSpec((hh, w), lambda i: (0, 0))


def kernel(x, edge_index, W0_self, W0_neigh, W1_self, W1_neigh, W_out):
    f32 = jnp.float32
    n, d = x.shape
    e = edge_index.shape[1]
    h = W0_self.shape[1]
    c = W_out.shape[1]
    nw = _NSC * _NTILES
    assert e % (nw * _CHUNK) == 0 and n % _NTILES == 0
    n_chunks = e // (nw * _CHUNK)

    src1 = edge_index[0]
    dst1 = edge_index[1]

    rb = 2000 if n % 2000 == 0 else n // 8
    grid = n // rb

    n_pad = -(-n // (_NTILES * 80)) * (_NTILES * 80)
    dgA, dgB = _make_deg(n_pad, h, n_chunks)(dst1)

    s0, t0 = pl.pallas_call(
        _l0_body,
        grid=(grid,),
        in_specs=[_row_spec(rb, d), _full_spec(d, h), _full_spec(d, h)],
        out_specs=[_row_spec(rb, h), _row_spec(rb, h)],
        out_shape=[jax.ShapeDtypeStruct((n, h), f32),
                   jax.ShapeDtypeStruct((n, h), f32)],
    )(x, W0_self, W0_neigh)

    p0L, p0R = _make_agg(n, h, n_chunks)(t0, src1, dst1)

    s1, t1 = pl.pallas_call(
        _mid_body,
        grid=(grid,),
        in_specs=[_row_spec(rb, h), _row_spec(rb, h), _row_spec(rb, h),
                  _row_spec(rb, h), _row_spec(rb, h),
                  _full_spec(h, h), _full_spec(h, h),
                  _full_spec(h, h), _full_spec(h, h)],
        out_specs=[_row_spec(rb, h), _row_spec(rb, h)],
        out_shape=[jax.ShapeDtypeStruct((n, h), f32),
                   jax.ShapeDtypeStruct((n, h), f32)],
    )(s0, p0L, p0R, dgA, dgB,
      W1_self[:h], W1_self[h:], W1_neigh[:h], W1_neigh[h:])

    p1L, p1R = _make_agg(n, h, n_chunks)(t1, src1, dst1)

    out = pl.pallas_call(
        _out_body,
        grid=(grid,),
        in_specs=[_row_spec(rb, h), _row_spec(rb, h), _row_spec(rb, h),
                  _row_spec(rb, h), _row_spec(rb, h),
                  _full_spec(h, c), _full_spec(h, c)],
        out_specs=_row_spec(rb, c),
        out_shape=jax.ShapeDtypeStruct((n, c), f32),
    )(s1, p1L, p1R, dgA, dgB, W_out[:h], W_out[h:])

    return out
